# Initial kernel scaffold; baseline (speedup 1.0000x reference)
import jax
import jax.numpy as jnp
from jax.experimental import pallas as pl


def kernel(x, edge_index, edge_type, basis, att, root, bias):
    raise NotImplementedError("write your pallas kernel here")


# speedup vs baseline: 9.7276x; 9.7276x over previous
import functools

import jax
import jax.numpy as jnp
from jax import lax
from jax.experimental import pallas as pl
from jax.experimental.pallas import tpu as pltpu
from jax.experimental.pallas import tpu_sc as plsc

N = 10000
E = 320000
F_IN = 128
F_OUT = 64
R = 2
NB = 30

NC = 2
NS = 16
NW = NC * NS
CHUNK = 128
CPT = 80
EPT = CPT * CHUNK
E_PAD = NW * EPT
ROWS2D = E_PAD // CHUNK
NBUF = 4
ACC_ROWS = 10240
ZROWS = ACC_ROWS // NS


def _mm_body(att_ref, x_ref, basis_ref, h_ref):
    x = x_ref[...]
    for r in range(R):
        w_r = att_ref[r, 0] * basis_ref[0]
        for b in range(1, NB):
            w_r = w_r + att_ref[r, b] * basis_ref[b]
        h_ref[r * N:(r + 1) * N, :] = jnp.dot(
            x, w_r, preferred_element_type=jnp.float32)


def _build_table(att, x, basis):
    return pl.pallas_call(
        _mm_body,
        out_shape=jax.ShapeDtypeStruct((R * N, F_OUT), jnp.float32),
        in_specs=[
            pl.BlockSpec(memory_space=pltpu.SMEM),
            pl.BlockSpec(memory_space=pltpu.VMEM),
            pl.BlockSpec(memory_space=pltpu.VMEM),
        ],
    )(att, x, basis)


def _edge_body(src_hbm, typ_hbm, dst_hbm, h_hbm, zeros_hbm, out_hbm,
               src_v, typ_v, dst_v, gidx_v, rows_v, acc,
               sem0, sem1, sem2, sem3):
    c = lax.axis_index("c")
    s = lax.axis_index("s")
    w = s * NC + c
    base = w * CPT

    pltpu.sync_copy(zeros_hbm, acc.at[pl.ds(s * ZROWS, ZROWS)])

    pltpu.sync_copy(src_hbm.at[pl.ds(base, CPT)], src_v)
    pltpu.sync_copy(typ_hbm.at[pl.ds(base, CPT)], typ_v)
    pltpu.sync_copy(dst_hbm.at[pl.ds(base, CPT)], dst_v)

    def gidx_row(j, carry):
        for k in range(CHUNK // 16):
            sl = pl.ds(k * 16, 16)
            gidx_v[j, sl] = typ_v[j, sl] * N + src_v[j, sl]
        return carry

    lax.fori_loop(0, CPT, gidx_row, 0)

    plsc.subcore_barrier()

    sems = (sem0, sem1, sem2, sem3)

    def start(j, b):
        pltpu.async_copy(h_hbm.at[gidx_v.at[j]], rows_v.at[b], sems[b])

    def finish(j, b):
        pltpu.make_async_copy(
            h_hbm.at[gidx_v.at[j]], rows_v.at[b], sems[b]).wait()
        pltpu.sync_copy(rows_v.at[b], acc.at[dst_v.at[j]], add=True)

    for b in range(NBUF):
        start(b, b)

    def ring(o, carry):
        for b in range(NBUF):
            j = o * NBUF + b
            finish(j, b)
            start(j + NBUF, b)
        return carry

    lax.fori_loop(0, CPT // NBUF - 1, ring, 0)
    for b in range(NBUF):
        finish((CPT // NBUF - 1) * NBUF + b, b)

    plsc.subcore_barrier()
    pltpu.sync_copy(acc.at[pl.ds(s * ZROWS, ZROWS)],
                    out_hbm.at[c].at[pl.ds(s * ZROWS, ZROWS)])


@functools.cache
def _edge_call():
    return pl.kernel(
        _edge_body,
        out_type=jax.ShapeDtypeStruct((NC, ACC_ROWS, F_OUT), jnp.float32),
        mesh=plsc.VectorSubcoreMesh(core_axis_name="c", subcore_axis_name="s",
                                    num_cores=NC, num_subcores=NS),
        compiler_params=pltpu.CompilerParams(use_tc_tiling_on_sc=False),
        scratch_types=[
            pltpu.VMEM((CPT, CHUNK), jnp.int32),
            pltpu.VMEM((CPT, CHUNK), jnp.int32),
            pltpu.VMEM((CPT, CHUNK), jnp.int32),
            pltpu.VMEM((CPT, CHUNK), jnp.int32),
            pltpu.VMEM((NBUF, CHUNK, F_OUT), jnp.float32),
            pltpu.VMEM_SHARED((ACC_ROWS, F_OUT), jnp.float32),
            pltpu.SemaphoreType.DMA,
            pltpu.SemaphoreType.DMA,
            pltpu.SemaphoreType.DMA,
            pltpu.SemaphoreType.DMA,
        ],
    )


def _combine_body(p_ref, x_ref, root_ref, bias_ref, o_ref):
    o_ref[...] = (p_ref[0, :N] + p_ref[1, :N]
                  + jnp.dot(x_ref[...], root_ref[...],
                            preferred_element_type=jnp.float32)
                  + bias_ref[...])


def _combine(partials, x, root, bias2d):
    return pl.pallas_call(
        _combine_body,
        out_shape=jax.ShapeDtypeStruct((N, F_OUT), jnp.float32),
    )(partials, x, root, bias2d)


def kernel(x, edge_index, edge_type, basis, att, root, bias):
    src = edge_index[0]
    dst = edge_index[1]
    pad = E_PAD - E
    src_p = jnp.concatenate([src, jnp.zeros((pad,), jnp.int32)])
    typ_p = jnp.concatenate([edge_type, jnp.zeros((pad,), jnp.int32)])
    dst_p = jnp.concatenate(
        [dst, N + (jnp.arange(pad, dtype=jnp.int32) % (ACC_ROWS - N))])
    src2d = src_p.reshape(ROWS2D, CHUNK)
    typ2d = typ_p.reshape(ROWS2D, CHUNK)
    dst2d = dst_p.reshape(ROWS2D, CHUNK)
    zeros = jnp.zeros((ZROWS, F_OUT), jnp.float32)

    table = _build_table(att, x, basis)
    partials = _edge_call()(src2d, typ2d, dst2d, table, zeros)
    return _combine(partials, x, root, bias.reshape(1, F_OUT))
